# baseline (speedup 1.0000x reference)
import functools

import jax
import jax.numpy as jnp
from jax import lax
from jax.experimental import pallas as pl
from jax.experimental.pallas import tpu as pltpu
from jax.experimental.pallas import tpu_sc as plsc

LANES = 128
R = 16
TB = 512


def _sc_gather(x_seq, phase, emb_i, phase_emb):
    B, L = x_seq.shape
    W = emb_i.shape[1]
    E = phase_emb.shape[1]
    info = plsc.get_sparse_core_info()
    NW = info.num_cores * info.num_subcores
    seq_w = B // NW
    nch = seq_w // R
    rows_ch = R * L
    ph_w = B // NW
    ph_st = ph_w // LANES

    mesh = plsc.VectorSubcoreMesh(core_axis_name="c", subcore_axis_name="s")

    @functools.partial(
        pl.kernel,
        mesh=mesh,
        out_type=(
            jax.ShapeDtypeStruct((B * L, W), jnp.int32),
            jax.ShapeDtypeStruct((B, E), jnp.float32),
        ),
        scratch_types=[
            pltpu.VMEM((2, R, L), jnp.int32),
            pltpu.VMEM((2, rows_ch, W), jnp.int32),
            pltpu.VMEM((ph_st, LANES), jnp.int32),
            pltpu.VMEM((ph_w, E), jnp.float32),
            pltpu.SemaphoreType.DMA,
            pltpu.SemaphoreType.DMA,
            pltpu.SemaphoreType.DMA,
            pltpu.SemaphoreType.DMA,
        ],
        compiler_params=pltpu.CompilerParams(use_tc_tiling_on_sc=False),
    )
    def k(xs_hbm, ph_hbm, emb_hbm, pemb_hbm, out_hbm, pout_hbm,
          idx_v, rows_v, pidx_v, ph_rows_v, gsem0, gsem1, wsem0, wsem1):
        wid = lax.axis_index("s") * info.num_cores + lax.axis_index("c")
        srow0 = wid * seq_w
        row0 = srow0 * L
        gsems = (gsem0, gsem1)
        wsems = (wsem0, wsem1)

        def fire(g, buf):
            s0 = pl.multiple_of(srow0 + g * R, 8)
            pltpu.sync_copy(xs_hbm.at[pl.ds(s0, R)], idx_v.at[buf])
            for j in range(R):
                pltpu.async_copy(
                    emb_hbm.at[idx_v.at[buf, j]],
                    rows_v.at[buf, pl.ds(j * L, L)],
                    gsems[buf],
                )

        def drain_gather(buf):
            for j in range(R):
                pltpu.make_async_copy(
                    emb_hbm.at[idx_v.at[buf, j]],
                    rows_v.at[buf, pl.ds(j * L, L)],
                    gsems[buf],
                ).wait()

        def write(g, buf):
            o0 = pl.multiple_of(row0 + g * rows_ch, 8)
            return pltpu.async_copy(
                rows_v.at[buf], out_hbm.at[pl.ds(o0, rows_ch)], wsems[buf])

        def wait_write(g, buf):
            o0 = pl.multiple_of(row0 + g * rows_ch, 8)
            pltpu.make_async_copy(
                rows_v.at[buf], out_hbm.at[pl.ds(o0, rows_ch)],
                wsems[buf]).wait()

        fire(0, 0)

        def body(i, _):
            for b in (0, 1):
                g = 2 * i + b
                nxt = g + 1

                @pl.when(nxt < nch)
                def _():
                    @pl.when(nxt >= 2)
                    def _():
                        wait_write(nxt - 2, 1 - b)
                    fire(nxt, 1 - b)

                drain_gather(b)
                write(g, b)
            return 0

        lax.fori_loop(0, nch // 2, body, 0)
        wait_write(nch - 2, 0)
        wait_write(nch - 1, 1)

        p0 = wid * ph_w
        for j in range(ph_st):
            pltpu.sync_copy(
                ph_hbm.at[pl.ds(pl.multiple_of(p0 + j * LANES, 8), LANES)],
                pidx_v.at[j])
        cps = [
            pltpu.async_copy(
                pemb_hbm.at[pidx_v.at[j]],
                ph_rows_v.at[pl.ds(j * LANES, LANES)],
                gsem0,
            )
            for j in range(ph_st)
        ]
        for cp in cps:
            cp.wait()
        pltpu.sync_copy(ph_rows_v,
                        pout_hbm.at[pl.ds(pl.multiple_of(p0, 8), ph_w)])

    return k(x_seq, phase, emb_i, phase_emb)


def _tc_mlp_body(g_ref, ph_ref, w1a_ref, w1p_ref, b1_ref, wh_ref, bh_ref,
                 o_ref):
    h = jnp.dot(g_ref[...], w1a_ref[...], preferred_element_type=jnp.float32)
    h += jnp.dot(ph_ref[...], w1p_ref[...], preferred_element_type=jnp.float32)
    h += b1_ref[...]
    h = jnp.maximum(h, 0.0)
    o_ref[...] = (
        jnp.dot(h, wh_ref[...], preferred_element_type=jnp.float32)
        + bh_ref[...]
    )


def kernel(x_seq, phase, emb, phase_emb, W1, b1, Wt, bt, Wp, bp):
    B, L = x_seq.shape
    V, E = emb.shape
    H = W1.shape[1]

    emb_i = jax.lax.bitcast_convert_type(
        emb.astype(jnp.bfloat16).reshape(V, E // 2, 2), jnp.int32)

    gathered, ph_rows = _sc_gather(x_seq.astype(jnp.int32),
                                   phase.astype(jnp.int32), emb_i, phase_emb)
    g2 = jax.lax.bitcast_convert_type(
        gathered, jnp.bfloat16).reshape(B, L * E)

    W1a = W1[: L * E].astype(jnp.bfloat16)
    W1p = W1[L * E:]
    w_head = jnp.pad(jnp.concatenate([Wt, Wp], axis=1), ((0, 0), (0, 3)))
    b_head = jnp.pad(jnp.concatenate([bt, bp]), (0, 3)).reshape(1, 8)
    b1r = b1.reshape(1, H)

    res = pl.pallas_call(
        _tc_mlp_body,
        grid=(B // TB,),
        in_specs=[
            pl.BlockSpec((TB, L * E), lambda i: (i, 0)),
            pl.BlockSpec((TB, E), lambda i: (i, 0)),
            pl.BlockSpec((L * E, H), lambda i: (0, 0)),
            pl.BlockSpec((E, H), lambda i: (0, 0)),
            pl.BlockSpec((1, H), lambda i: (0, 0)),
            pl.BlockSpec((H, 8), lambda i: (0, 0)),
            pl.BlockSpec((1, 8), lambda i: (0, 0)),
        ],
        out_specs=pl.BlockSpec((TB, 8), lambda i: (i, 0)),
        out_shape=jax.ShapeDtypeStruct((B, 8), jnp.float32),
    )(g2, ph_rows, W1a, W1p, b1r, w_head, b_head)

    return res[:, :4], res[:, 4:5]


# speedup vs baseline: 14.9934x; 14.9934x over previous
import functools

import jax
import jax.numpy as jnp
from jax import lax
from jax.experimental import pallas as pl
from jax.experimental.pallas import tpu as pltpu
from jax.experimental.pallas import tpu_sc as plsc

LANES = 128
R = 16
TB = 512


def _sc_gather(x_seq4, phase, emb4, phase_emb):
    B, L = x_seq4.shape
    E = emb4.shape[1]
    info = plsc.get_sparse_core_info()
    NW = info.num_cores * info.num_subcores
    seq_w = B // NW
    nch = seq_w // R
    rows_ch = R * L
    ph_w = B // NW
    ph_st = ph_w // LANES

    mesh = plsc.VectorSubcoreMesh(core_axis_name="c", subcore_axis_name="s")

    @functools.partial(
        pl.kernel,
        mesh=mesh,
        out_type=(
            jax.ShapeDtypeStruct((B * L, E), jnp.float32),
            jax.ShapeDtypeStruct((B, E), jnp.float32),
        ),
        scratch_types=[
            pltpu.VMEM((2, R, L), jnp.int32),
            pltpu.VMEM((2, rows_ch, E), jnp.float32),
            pltpu.VMEM((ph_st, LANES), jnp.int32),
            pltpu.VMEM((ph_w, E), jnp.float32),
            pltpu.SemaphoreType.DMA,
            pltpu.SemaphoreType.DMA,
            pltpu.SemaphoreType.DMA,
            pltpu.SemaphoreType.DMA,
        ],
        compiler_params=pltpu.CompilerParams(use_tc_tiling_on_sc=False),
    )
    def k(xs_hbm, ph_hbm, emb_hbm, pemb_hbm, out_hbm, pout_hbm,
          idx_v, rows_v, pidx_v, ph_rows_v, gsem0, gsem1, wsem0, wsem1):
        wid = lax.axis_index("s") * info.num_cores + lax.axis_index("c")
        srow0 = wid * seq_w
        row0 = srow0 * L
        gsems = (gsem0, gsem1)
        wsems = (wsem0, wsem1)

        def fire(g, buf):
            s0 = pl.multiple_of(srow0 + g * R, 8)
            pltpu.sync_copy(xs_hbm.at[pl.ds(s0, R)], idx_v.at[buf])
            for j in range(R):
                pltpu.async_copy(
                    emb_hbm.at[idx_v.at[buf, j]],
                    rows_v.at[buf, pl.ds(j * L, L)],
                    gsems[buf],
                )

        def drain_gather(buf):
            for j in range(R):
                pltpu.make_async_copy(
                    emb_hbm.at[idx_v.at[buf, j]],
                    rows_v.at[buf, pl.ds(j * L, L)],
                    gsems[buf],
                ).wait()

        def write(g, buf):
            o0 = pl.multiple_of(row0 + g * rows_ch, 8)
            return pltpu.async_copy(
                rows_v.at[buf], out_hbm.at[pl.ds(o0, rows_ch)], wsems[buf])

        def wait_write(g, buf):
            o0 = pl.multiple_of(row0 + g * rows_ch, 8)
            pltpu.make_async_copy(
                rows_v.at[buf], out_hbm.at[pl.ds(o0, rows_ch)],
                wsems[buf]).wait()

        fire(0, 0)

        def body(i, _):
            for b in (0, 1):
                g = 2 * i + b
                nxt = g + 1

                @pl.when(nxt < nch)
                def _():
                    @pl.when(nxt >= 2)
                    def _():
                        wait_write(nxt - 2, 1 - b)
                    fire(nxt, 1 - b)

                drain_gather(b)
                write(g, b)
            return 0

        lax.fori_loop(0, nch // 2, body, 0)
        wait_write(nch - 2, 0)
        wait_write(nch - 1, 1)

        p0 = wid * ph_w
        for j in range(ph_st):
            pltpu.sync_copy(
                ph_hbm.at[pl.ds(pl.multiple_of(p0 + j * LANES, 8), LANES)],
                pidx_v.at[j])
        cps = [
            pltpu.async_copy(
                pemb_hbm.at[pidx_v.at[j]],
                ph_rows_v.at[pl.ds(j * LANES, LANES)],
                gsem0,
            )
            for j in range(ph_st)
        ]
        for cp in cps:
            cp.wait()
        pltpu.sync_copy(ph_rows_v,
                        pout_hbm.at[pl.ds(pl.multiple_of(p0, 8), ph_w)])

    return k(x_seq4, phase, emb4, phase_emb)


def _tc_mlp_body(g_ref, ph_ref, w1a_ref, w1p_ref, b1_ref, wh_ref, bh_ref,
                 o_ref):
    h = jnp.dot(g_ref[...], w1a_ref[...], preferred_element_type=jnp.float32)
    h += jnp.dot(ph_ref[...], w1p_ref[...], preferred_element_type=jnp.float32)
    h += b1_ref[...]
    h = jnp.maximum(h, 0.0)
    o_ref[...] = (
        jnp.dot(h, wh_ref[...], preferred_element_type=jnp.float32)
        + bh_ref[...]
    )


def kernel(x_seq, phase, emb, phase_emb, W1, b1, Wt, bt, Wp, bp):
    B, L = x_seq.shape
    V, E = emb.shape
    H = W1.shape[1]

    emb4 = jnp.pad(emb, ((0, 0), (0, 128 - E))).reshape(4 * V, E)

    gathered, ph_rows = _sc_gather(x_seq.astype(jnp.int32) * 4,
                                   phase.astype(jnp.int32), emb4, phase_emb)
    g2 = gathered.reshape(B, L * E)

    W1a = W1[: L * E]
    W1p = W1[L * E:]
    w_head = jnp.pad(jnp.concatenate([Wt, Wp], axis=1), ((0, 0), (0, 3)))
    b_head = jnp.pad(jnp.concatenate([bt, bp]), (0, 3)).reshape(1, 8)
    b1r = b1.reshape(1, H)

    res = pl.pallas_call(
        _tc_mlp_body,
        grid=(B // TB,),
        in_specs=[
            pl.BlockSpec((TB, L * E), lambda i: (i, 0)),
            pl.BlockSpec((TB, E), lambda i: (i, 0)),
            pl.BlockSpec((L * E, H), lambda i: (0, 0)),
            pl.BlockSpec((E, H), lambda i: (0, 0)),
            pl.BlockSpec((1, H), lambda i: (0, 0)),
            pl.BlockSpec((H, 8), lambda i: (0, 0)),
            pl.BlockSpec((1, 8), lambda i: (0, 0)),
        ],
        out_specs=pl.BlockSpec((TB, 8), lambda i: (i, 0)),
        out_shape=jax.ShapeDtypeStruct((B, 8), jnp.float32),
    )(g2, ph_rows, W1a, W1p, b1r, w_head, b_head)

    return res[:, :4], res[:, 4:5]


# speedup vs baseline: 19.2187x; 1.2818x over previous
import functools

import jax
import jax.numpy as jnp
from jax import lax
from jax.experimental import pallas as pl
from jax.experimental.pallas import tpu as pltpu
from jax.experimental.pallas import tpu_sc as plsc

LANES = 128
R = 16
TB = 512
CB = 2048


def _tc_repack(embT):
    E, V = embT.shape
    nblk = (V + 4 * CB - 1) // (4 * CB)
    ro = nblk * CB

    def body(i0, i1, i2, i3, o_ref):
        for k, r in enumerate((i0, i1, i2, i3)):
            o_ref[:, k * E:(k + 1) * E] = r[...].T

    last = (V - 1) // CB
    in_specs = [
        pl.BlockSpec((E, CB),
                     (lambda i, k=k: (0, jnp.minimum(4 * i + k, last))))
        for k in range(4)
    ]
    return pl.pallas_call(
        body,
        grid=(nblk,),
        in_specs=in_specs,
        out_specs=pl.BlockSpec((CB, 4 * E), lambda i: (i, 0)),
        out_shape=jax.ShapeDtypeStruct((ro, 4 * E), jnp.float32),
    )(embT, embT, embT, embT)


def _sc_gather(x_seq4, phase, emb4, phase_emb):
    B, L = x_seq4.shape
    E = emb4.shape[1]
    info = plsc.get_sparse_core_info()
    NW = info.num_cores * info.num_subcores
    seq_w = B // NW
    nch = seq_w // R
    rows_ch = R * L
    ph_w = B // NW
    ph_st = ph_w // LANES

    mesh = plsc.VectorSubcoreMesh(core_axis_name="c", subcore_axis_name="s")

    @functools.partial(
        pl.kernel,
        mesh=mesh,
        out_type=(
            jax.ShapeDtypeStruct((B * L, E), jnp.float32),
            jax.ShapeDtypeStruct((B, E), jnp.float32),
        ),
        scratch_types=[
            pltpu.VMEM((2, R, L), jnp.int32),
            pltpu.VMEM((2, rows_ch, E), jnp.float32),
            pltpu.VMEM((ph_st, LANES), jnp.int32),
            pltpu.VMEM((ph_w, E), jnp.float32),
            pltpu.SemaphoreType.DMA,
            pltpu.SemaphoreType.DMA,
            pltpu.SemaphoreType.DMA,
            pltpu.SemaphoreType.DMA,
        ],
        compiler_params=pltpu.CompilerParams(use_tc_tiling_on_sc=False),
    )
    def k(xs_hbm, ph_hbm, emb_hbm, pemb_hbm, out_hbm, pout_hbm,
          idx_v, rows_v, pidx_v, ph_rows_v, gsem0, gsem1, wsem0, wsem1):
        wid = lax.axis_index("s") * info.num_cores + lax.axis_index("c")
        srow0 = wid * seq_w
        row0 = srow0 * L
        gsems = (gsem0, gsem1)
        wsems = (wsem0, wsem1)

        def fire(g, buf):
            s0 = pl.multiple_of(srow0 + g * R, 8)
            pltpu.sync_copy(xs_hbm.at[pl.ds(s0, R)], idx_v.at[buf])
            for j in range(R):
                pltpu.async_copy(
                    emb_hbm.at[idx_v.at[buf, j]],
                    rows_v.at[buf, pl.ds(j * L, L)],
                    gsems[buf],
                )

        def drain_gather(buf):
            for j in range(R):
                pltpu.make_async_copy(
                    emb_hbm.at[idx_v.at[buf, j]],
                    rows_v.at[buf, pl.ds(j * L, L)],
                    gsems[buf],
                ).wait()

        def write(g, buf):
            o0 = pl.multiple_of(row0 + g * rows_ch, 8)
            return pltpu.async_copy(
                rows_v.at[buf], out_hbm.at[pl.ds(o0, rows_ch)], wsems[buf])

        def wait_write(g, buf):
            o0 = pl.multiple_of(row0 + g * rows_ch, 8)
            pltpu.make_async_copy(
                rows_v.at[buf], out_hbm.at[pl.ds(o0, rows_ch)],
                wsems[buf]).wait()

        fire(0, 0)

        def body(i, _):
            for b in (0, 1):
                g = 2 * i + b
                nxt = g + 1

                @pl.when(nxt < nch)
                def _():
                    @pl.when(nxt >= 2)
                    def _():
                        wait_write(nxt - 2, 1 - b)
                    fire(nxt, 1 - b)

                drain_gather(b)
                write(g, b)
            return 0

        lax.fori_loop(0, nch // 2, body, 0)
        wait_write(nch - 2, 0)
        wait_write(nch - 1, 1)

        p0 = wid * ph_w
        for j in range(ph_st):
            pltpu.sync_copy(
                ph_hbm.at[pl.ds(pl.multiple_of(p0 + j * LANES, 8), LANES)],
                pidx_v.at[j])
        cps = [
            pltpu.async_copy(
                pemb_hbm.at[pidx_v.at[j]],
                ph_rows_v.at[pl.ds(j * LANES, LANES)],
                gsem0,
            )
            for j in range(ph_st)
        ]
        for cp in cps:
            cp.wait()
        pltpu.sync_copy(ph_rows_v,
                        pout_hbm.at[pl.ds(pl.multiple_of(p0, 8), ph_w)])

    return k(x_seq4, phase, emb4, phase_emb)


def _tc_mlp_body(g_ref, ph_ref, w1a_ref, w1p_ref, b1_ref, wh_ref, bh_ref,
                 o_ref):
    h = jnp.dot(g_ref[...], w1a_ref[...], preferred_element_type=jnp.float32)
    h += jnp.dot(ph_ref[...], w1p_ref[...], preferred_element_type=jnp.float32)
    h += b1_ref[...]
    h = jnp.maximum(h, 0.0)
    o_ref[...] = (
        jnp.dot(h, wh_ref[...], preferred_element_type=jnp.float32)
        + bh_ref[...]
    )


def kernel(x_seq, phase, emb, phase_emb, W1, b1, Wt, bt, Wp, bp):
    B, L = x_seq.shape
    V, E = emb.shape
    H = W1.shape[1]

    embC = _tc_repack(emb.T)
    emb4 = embC.reshape(embC.shape[0] * 4, E)
    v = x_seq.astype(jnp.int32)
    xq = (((v >> 13) << 13) | ((v & 2047) << 2) | ((v >> 11) & 3))

    gathered, ph_rows = _sc_gather(xq, phase.astype(jnp.int32),
                                   emb4, phase_emb)
    g2 = gathered.reshape(B, L * E)

    W1a = W1[: L * E]
    W1p = W1[L * E:]
    w_head = jnp.pad(jnp.concatenate([Wt, Wp], axis=1), ((0, 0), (0, 3)))
    b_head = jnp.pad(jnp.concatenate([bt, bp]), (0, 3)).reshape(1, 8)
    b1r = b1.reshape(1, H)

    res = pl.pallas_call(
        _tc_mlp_body,
        grid=(B // TB,),
        in_specs=[
            pl.BlockSpec((TB, L * E), lambda i: (i, 0)),
            pl.BlockSpec((TB, E), lambda i: (i, 0)),
            pl.BlockSpec((L * E, H), lambda i: (0, 0)),
            pl.BlockSpec((E, H), lambda i: (0, 0)),
            pl.BlockSpec((1, H), lambda i: (0, 0)),
            pl.BlockSpec((H, 8), lambda i: (0, 0)),
            pl.BlockSpec((1, 8), lambda i: (0, 0)),
        ],
        out_specs=pl.BlockSpec((TB, 8), lambda i: (i, 0)),
        out_shape=jax.ShapeDtypeStruct((B, 8), jnp.float32),
    )(g2, ph_rows, W1a, W1p, b1r, w_head, b_head)

    return res[:, :4], res[:, 4:5]


# speedup vs baseline: 20.1181x; 1.0468x over previous
import functools

import jax
import jax.numpy as jnp
from jax import lax
from jax.experimental import pallas as pl
from jax.experimental.pallas import tpu as pltpu
from jax.experimental.pallas import tpu_sc as plsc

LANES = 128
R = 16
TB = 512
CB = 2048


def _tc_repack(embT):
    E, V = embT.shape
    nblk = (V + 4 * CB - 1) // (4 * CB)
    ro = nblk * CB

    def body(i0, i1, i2, i3, o_ref):
        for k, r in enumerate((i0, i1, i2, i3)):
            o_ref[:, k * E:(k + 1) * E] = r[...].T

    last = (V - 1) // CB
    in_specs = [
        pl.BlockSpec((E, CB),
                     (lambda i, k=k: (0, jnp.minimum(4 * i + k, last))))
        for k in range(4)
    ]
    return pl.pallas_call(
        body,
        grid=(nblk,),
        in_specs=in_specs,
        out_specs=pl.BlockSpec((CB, 4 * E), lambda i: (i, 0)),
        out_shape=jax.ShapeDtypeStruct((ro, 4 * E), jnp.float32),
    )(embT, embT, embT, embT)


def _sc_gather(x_seq4, phase, emb4, phase_emb):
    B, L = x_seq4.shape
    E = emb4.shape[1]
    info = plsc.get_sparse_core_info()
    NW = info.num_cores * info.num_subcores
    seq_w = B // NW
    nch = seq_w // R
    rows_ch = R * L
    ph_w = B // NW
    ph_st = ph_w // LANES

    mesh = plsc.VectorSubcoreMesh(core_axis_name="c", subcore_axis_name="s")

    @functools.partial(
        pl.kernel,
        mesh=mesh,
        out_type=(
            jax.ShapeDtypeStruct((B * L, E), jnp.float32),
            jax.ShapeDtypeStruct((B, E), jnp.float32),
        ),
        scratch_types=[
            pltpu.VMEM((2, R, L), jnp.int32),
            pltpu.VMEM((2, rows_ch, E), jnp.float32),
            pltpu.VMEM((ph_st, LANES), jnp.int32),
            pltpu.VMEM((ph_w, E), jnp.float32),
            pltpu.SemaphoreType.DMA,
            pltpu.SemaphoreType.DMA,
            pltpu.SemaphoreType.DMA,
            pltpu.SemaphoreType.DMA,
        ],
        compiler_params=pltpu.CompilerParams(use_tc_tiling_on_sc=False),
    )
    def k(xs_hbm, ph_hbm, emb_hbm, pemb_hbm, out_hbm, pout_hbm,
          idx_v, rows_v, pidx_v, ph_rows_v, gsem0, gsem1, wsem0, wsem1):
        wid = lax.axis_index("s") * info.num_cores + lax.axis_index("c")
        srow0 = wid * seq_w
        row0 = srow0 * L
        gsems = (gsem0, gsem1)
        wsems = (wsem0, wsem1)

        def fire(g, buf):
            s0 = pl.multiple_of(srow0 + g * R, 8)
            pltpu.sync_copy(xs_hbm.at[pl.ds(s0, R)], idx_v.at[buf])
            for j in range(R):
                pltpu.async_copy(
                    emb_hbm.at[idx_v.at[buf, j]],
                    rows_v.at[buf, pl.ds(j * L, L)],
                    gsems[buf],
                )

        def drain_gather(buf):
            for j in range(R):
                pltpu.make_async_copy(
                    emb_hbm.at[idx_v.at[buf, j]],
                    rows_v.at[buf, pl.ds(j * L, L)],
                    gsems[buf],
                ).wait()

        def write(g, buf):
            o0 = pl.multiple_of(row0 + g * rows_ch, 8)
            return pltpu.async_copy(
                rows_v.at[buf], out_hbm.at[pl.ds(o0, rows_ch)], wsems[buf])

        def wait_write(g, buf):
            o0 = pl.multiple_of(row0 + g * rows_ch, 8)
            pltpu.make_async_copy(
                rows_v.at[buf], out_hbm.at[pl.ds(o0, rows_ch)],
                wsems[buf]).wait()

        fire(0, 0)

        def body(i, _):
            for b in (0, 1):
                g = 2 * i + b
                nxt = g + 1

                @pl.when(nxt < nch)
                def _():
                    @pl.when(nxt >= 2)
                    def _():
                        wait_write(nxt - 2, 1 - b)
                    fire(nxt, 1 - b)

                drain_gather(b)
                write(g, b)
            return 0

        lax.fori_loop(0, nch // 2, body, 0)
        wait_write(nch - 2, 0)
        wait_write(nch - 1, 1)

        p0 = wid * ph_w
        for j in range(ph_st):
            pltpu.sync_copy(
                ph_hbm.at[pl.ds(pl.multiple_of(p0 + j * LANES, 8), LANES)],
                pidx_v.at[j])
        cps = [
            pltpu.async_copy(
                pemb_hbm.at[pidx_v.at[j]],
                ph_rows_v.at[pl.ds(j * LANES, LANES)],
                gsem0,
            )
            for j in range(ph_st)
        ]
        for cp in cps:
            cp.wait()
        pltpu.sync_copy(ph_rows_v,
                        pout_hbm.at[pl.ds(pl.multiple_of(p0, 8), ph_w)])

    return k(x_seq4, phase, emb4, phase_emb)


def _tc_mlp_body(g_ref, ph_ref, w1a_ref, w1p_ref, b1_ref, wh_ref, bh_ref,
                 o_ref):
    h = jnp.dot(g_ref[...], w1a_ref[...], preferred_element_type=jnp.float32)
    h += jnp.dot(ph_ref[...], w1p_ref[...], preferred_element_type=jnp.float32)
    h += b1_ref[...]
    h = jnp.maximum(h, 0.0)
    o_ref[...] = (
        jnp.dot(h, wh_ref[...], preferred_element_type=jnp.float32)
        + bh_ref[...]
    )


def kernel(x_seq, phase, emb, phase_emb, W1, b1, Wt, bt, Wp, bp):
    B, L = x_seq.shape
    V, E = emb.shape
    H = W1.shape[1]

    embC = _tc_repack(emb.T)
    emb4 = embC.reshape(embC.shape[0] * 4, E)
    v = x_seq.astype(jnp.int32)
    xq = (((v >> 13) << 13) | ((v & 2047) << 2) | ((v >> 11) & 3))
    ph32 = phase.astype(jnp.int32)

    W1a = W1[: L * E]
    W1p = W1[L * E:]
    w_head = jnp.pad(jnp.concatenate([Wt, Wp], axis=1), ((0, 0), (0, 3)))
    b_head = jnp.pad(jnp.concatenate([bt, bp]), (0, 3)).reshape(1, 8)
    b1r = b1.reshape(1, H)

    def mlp_half(xq_h, ph_h):
        bh = xq_h.shape[0]
        gathered, ph_rows = _sc_gather(xq_h, ph_h, emb4, phase_emb)
        g2 = gathered.reshape(bh, L * E)
        return pl.pallas_call(
            _tc_mlp_body,
            grid=(bh // TB,),
            in_specs=[
                pl.BlockSpec((TB, L * E), lambda i: (i, 0)),
                pl.BlockSpec((TB, E), lambda i: (i, 0)),
                pl.BlockSpec((L * E, H), lambda i: (0, 0)),
                pl.BlockSpec((E, H), lambda i: (0, 0)),
                pl.BlockSpec((1, H), lambda i: (0, 0)),
                pl.BlockSpec((H, 8), lambda i: (0, 0)),
                pl.BlockSpec((1, 8), lambda i: (0, 0)),
            ],
            out_specs=pl.BlockSpec((TB, 8), lambda i: (i, 0)),
            out_shape=jax.ShapeDtypeStruct((bh, 8), jnp.float32),
        )(g2, ph_rows, W1a, W1p, b1r, w_head, b_head)

    half = B // 2
    res = jnp.concatenate(
        [mlp_half(xq[:half], ph32[:half]), mlp_half(xq[half:], ph32[half:])])

    return res[:, :4], res[:, 4:5]


# speedup vs baseline: 20.2495x; 1.0065x over previous
import functools

import jax
import jax.numpy as jnp
from jax import lax
from jax.experimental import pallas as pl
from jax.experimental.pallas import tpu as pltpu
from jax.experimental.pallas import tpu_sc as plsc

LANES = 128
R = 16
TB = 512
CB = 2048


def _tc_repack(embT):
    E, V = embT.shape
    nblk = (V + 4 * CB - 1) // (4 * CB)
    ro = nblk * CB

    def body(i0, i1, i2, i3, o_ref):
        for k, r in enumerate((i0, i1, i2, i3)):
            o_ref[:, k * E:(k + 1) * E] = r[...].T

    last = (V - 1) // CB
    in_specs = [
        pl.BlockSpec((E, CB),
                     (lambda i, k=k: (0, jnp.minimum(4 * i + k, last))))
        for k in range(4)
    ]
    return pl.pallas_call(
        body,
        grid=(nblk,),
        in_specs=in_specs,
        out_specs=pl.BlockSpec((CB, 4 * E), lambda i: (i, 0)),
        out_shape=jax.ShapeDtypeStruct((ro, 4 * E), jnp.float32),
    )(embT, embT, embT, embT)


def _sc_gather(x_seq4, phase, emb4, phase_emb):
    B, L = x_seq4.shape
    E = emb4.shape[1]
    info = plsc.get_sparse_core_info()
    NW = info.num_cores * info.num_subcores
    seq_w = B // NW
    nch = seq_w // R
    rows_ch = R * L
    ph_w = B // NW
    ph_st = ph_w // LANES

    mesh = plsc.VectorSubcoreMesh(core_axis_name="c", subcore_axis_name="s")

    @functools.partial(
        pl.kernel,
        mesh=mesh,
        out_type=(
            jax.ShapeDtypeStruct((B * L, E), jnp.float32),
            jax.ShapeDtypeStruct((B, E), jnp.float32),
        ),
        scratch_types=[
            pltpu.VMEM((2, R, L), jnp.int32),
            pltpu.VMEM((2, rows_ch, E), jnp.float32),
            pltpu.VMEM((ph_st, LANES), jnp.int32),
            pltpu.VMEM((ph_w, E), jnp.float32),
            pltpu.SemaphoreType.DMA,
            pltpu.SemaphoreType.DMA,
            pltpu.SemaphoreType.DMA,
            pltpu.SemaphoreType.DMA,
        ],
        compiler_params=pltpu.CompilerParams(use_tc_tiling_on_sc=False),
    )
    def k(xs_hbm, ph_hbm, emb_hbm, pemb_hbm, out_hbm, pout_hbm,
          idx_v, rows_v, pidx_v, ph_rows_v, gsem0, gsem1, wsem0, wsem1):
        wid = lax.axis_index("s") * info.num_cores + lax.axis_index("c")
        srow0 = wid * seq_w
        row0 = srow0 * L
        gsems = (gsem0, gsem1)
        wsems = (wsem0, wsem1)

        def fire(g, buf):
            s0 = pl.multiple_of(srow0 + g * R, 8)
            pltpu.sync_copy(xs_hbm.at[pl.ds(s0, R)], idx_v.at[buf])
            for j in range(R):
                pltpu.async_copy(
                    emb_hbm.at[idx_v.at[buf, j]],
                    rows_v.at[buf, pl.ds(j * L, L)],
                    gsems[buf],
                )

        def drain_gather(buf):
            for j in range(R):
                pltpu.make_async_copy(
                    emb_hbm.at[idx_v.at[buf, j]],
                    rows_v.at[buf, pl.ds(j * L, L)],
                    gsems[buf],
                ).wait()

        def write(g, buf):
            o0 = pl.multiple_of(row0 + g * rows_ch, 8)
            return pltpu.async_copy(
                rows_v.at[buf], out_hbm.at[pl.ds(o0, rows_ch)], wsems[buf])

        def wait_write(g, buf):
            o0 = pl.multiple_of(row0 + g * rows_ch, 8)
            pltpu.make_async_copy(
                rows_v.at[buf], out_hbm.at[pl.ds(o0, rows_ch)],
                wsems[buf]).wait()

        fire(0, 0)

        def body(i, _):
            for b in (0, 1):
                g = 2 * i + b
                nxt = g + 1

                @pl.when(nxt < nch)
                def _():
                    @pl.when(nxt >= 2)
                    def _():
                        wait_write(nxt - 2, 1 - b)
                    fire(nxt, 1 - b)

                drain_gather(b)
                write(g, b)
            return 0

        lax.fori_loop(0, nch // 2, body, 0)
        wait_write(nch - 2, 0)
        wait_write(nch - 1, 1)

        p0 = wid * ph_w
        for j in range(ph_st):
            pltpu.sync_copy(
                ph_hbm.at[pl.ds(pl.multiple_of(p0 + j * LANES, 8), LANES)],
                pidx_v.at[j])
        cps = [
            pltpu.async_copy(
                pemb_hbm.at[pidx_v.at[j]],
                ph_rows_v.at[pl.ds(j * LANES, LANES)],
                gsem0,
            )
            for j in range(ph_st)
        ]
        for cp in cps:
            cp.wait()
        pltpu.sync_copy(ph_rows_v,
                        pout_hbm.at[pl.ds(pl.multiple_of(p0, 8), ph_w)])

    return k(x_seq4, phase, emb4, phase_emb)


def _tc_mlp_body(g_ref, ph_ref, w1a_ref, w1p_ref, b1_ref, wh_ref, bh_ref,
                 o_ref):
    h = jnp.dot(g_ref[...], w1a_ref[...], preferred_element_type=jnp.float32)
    h += jnp.dot(ph_ref[...], w1p_ref[...], preferred_element_type=jnp.float32)
    h += b1_ref[...]
    h = jnp.maximum(h, 0.0)
    o_ref[...] = (
        jnp.dot(h, wh_ref[...], preferred_element_type=jnp.float32)
        + bh_ref[...]
    )


def kernel(x_seq, phase, emb, phase_emb, W1, b1, Wt, bt, Wp, bp):
    B, L = x_seq.shape
    V, E = emb.shape
    H = W1.shape[1]

    embC = _tc_repack(emb.T)
    emb4 = embC.reshape(embC.shape[0] * 4, E)
    v = x_seq.astype(jnp.int32)
    xq = (((v >> 13) << 13) | ((v & 2047) << 2) | ((v >> 11) & 3))
    ph32 = phase.astype(jnp.int32)

    W1a = W1[: L * E]
    W1p = W1[L * E:]
    w_head = jnp.pad(jnp.concatenate([Wt, Wp], axis=1), ((0, 0), (0, 3)))
    b_head = jnp.pad(jnp.concatenate([bt, bp]), (0, 3)).reshape(1, 8)
    b1r = b1.reshape(1, H)

    def mlp_half(xq_h, ph_h):
        bh = xq_h.shape[0]
        gathered, ph_rows = _sc_gather(xq_h, ph_h, emb4, phase_emb)
        g2 = gathered.reshape(bh, L * E)
        return pl.pallas_call(
            _tc_mlp_body,
            grid=(bh // TB,),
            in_specs=[
                pl.BlockSpec((TB, L * E), lambda i: (i, 0)),
                pl.BlockSpec((TB, E), lambda i: (i, 0)),
                pl.BlockSpec((L * E, H), lambda i: (0, 0)),
                pl.BlockSpec((E, H), lambda i: (0, 0)),
                pl.BlockSpec((1, H), lambda i: (0, 0)),
                pl.BlockSpec((H, 8), lambda i: (0, 0)),
                pl.BlockSpec((1, 8), lambda i: (0, 0)),
            ],
            out_specs=pl.BlockSpec((TB, 8), lambda i: (i, 0)),
            out_shape=jax.ShapeDtypeStruct((bh, 8), jnp.float32),
        )(g2, ph_rows, W1a, W1p, b1r, w_head, b_head)

    q = B // 4
    res = jnp.concatenate(
        [mlp_half(xq[i * q:(i + 1) * q], ph32[i * q:(i + 1) * q])
         for i in range(4)])

    return res[:, :4], res[:, 4:5]
